# Initial kernel scaffold; baseline (speedup 1.0000x reference)
import jax
import jax.numpy as jnp
from jax.experimental import pallas as pl


def kernel(x, table, W, b):
    raise NotImplementedError("write your pallas kernel here")


# speedup vs baseline: 2.0744x; 2.0744x over previous
import dataclasses
import functools

import jax
import jax.numpy as jnp
from jax import lax
from jax.experimental import pallas as pl
from jax.experimental.pallas import tpu as pltpu
from jax.experimental.pallas import tpu_sc as plsc

B = 4096
L = 200
D = 32
NW = 32
BPW = B // NW
IPW = BPW * L
C0 = 128
C1 = L - C0


def _sc_kernel(x_hbm, table_hbm, wb_hbm, out_hbm, idx_v, rows_v, out_v,
               wb_v, sem):
    wid = lax.axis_index("s") * 2 + lax.axis_index("c")

    pltpu.sync_copy(wb_hbm, wb_v)
    pltpu.sync_copy(x_hbm.at[pl.ds(wid * IPW, IPW)], idx_v)

    w0 = wb_v[pl.ds(0, 16)]
    w1 = wb_v[pl.ds(16, 16)]
    biasv = wb_v[pl.ds(32, 16)]

    lane = lax.broadcasted_iota(jnp.int32, (16,), 0)

    @pl.loop(0, BPW // 16)
    def _grp(g):
        def row_body(j, vec):
            base = (g * 16 + j) * L
            c0 = pltpu.make_async_copy(
                table_hbm.at[idx_v.at[pl.ds(base, C0)]],
                rows_v.at[pl.ds(0, C0)], sem)
            c1 = pltpu.make_async_copy(
                table_hbm.at[idx_v.at[pl.ds(base + C0, C1)]],
                rows_v.at[pl.ds(C0, C1)], sem)
            c0.start()
            c1.start()
            c0.wait()
            c1.wait()

            def acc_body(l, carry):
                a0, a1 = carry
                a0 = a0 + rows_v[l, pl.ds(0, 16)]
                a1 = a1 + rows_v[l, pl.ds(16, 16)]
                return a0, a1

            zero = jnp.zeros((16,), jnp.float32)
            a0, a1 = lax.fori_loop(0, L, acc_body, (zero, zero), unroll=8)
            t = (a0 * w0 + a1 * w1) * (1.0 / L) + biasv
            return jnp.where(lane == j, jnp.sum(t), vec)

        vec = lax.fori_loop(0, 16, row_body, jnp.zeros((16,), jnp.float32))
        out_v[pl.ds(g * 16, 16)] = vec

    pltpu.sync_copy(out_v, out_hbm.at[pl.ds(wid * BPW, BPW)])


def kernel(x, table, W, b):
    xf = x.reshape(-1).astype(jnp.int32)
    wb = jnp.concatenate(
        [W.reshape(-1), jnp.broadcast_to(b / 16.0, (16,))]
    ).astype(jnp.float32)

    mesh = plsc.VectorSubcoreMesh(core_axis_name="c", subcore_axis_name="s")
    cp = pltpu.CompilerParams()
    fields = pltpu.CompilerParams.__dataclass_fields__
    if "needs_layout_passes" in fields:
        cp = dataclasses.replace(cp, needs_layout_passes=False)
    if "use_tc_tiling_on_sc" in fields:
        cp = dataclasses.replace(cp, use_tc_tiling_on_sc=False)
    run = functools.partial(
        pl.kernel,
        compiler_params=cp,
        out_type=jax.ShapeDtypeStruct((B,), jnp.float32),
        mesh=mesh,
        scratch_types=[
            pltpu.VMEM((IPW,), jnp.int32),
            pltpu.VMEM((L, D), jnp.float32),
            pltpu.VMEM((BPW,), jnp.float32),
            pltpu.VMEM((48,), jnp.float32),
            pltpu.SemaphoreType.DMA,
        ],
    )(_sc_kernel)

    out = run(xf, table, wb)
    return out.reshape(B, 1)


# speedup vs baseline: 2.4733x; 1.1923x over previous
import dataclasses
import functools

import jax
import jax.numpy as jnp
from jax import lax
from jax.experimental import pallas as pl
from jax.experimental.pallas import tpu as pltpu
from jax.experimental.pallas import tpu_sc as plsc

B = 4096
L = 200
D = 32
NW = 32
BPW = B // NW
IPW = BPW * L
C0 = 128
C1 = L - C0
NBUF = 16


def _sc_kernel(x_hbm, table_hbm, wb_hbm, out_hbm, idx_v, rows_v, out_v,
               wb_v, sem):
    wid = lax.axis_index("s") * 2 + lax.axis_index("c")

    pltpu.sync_copy(wb_hbm, wb_v)
    pltpu.sync_copy(x_hbm.at[pl.ds(wid * IPW, IPW)], idx_v)

    w0 = wb_v[pl.ds(0, 16)]
    w1 = wb_v[pl.ds(16, 16)]
    biasv = wb_v[pl.ds(32, 16)]

    lane = lax.broadcasted_iota(jnp.int32, (16,), 0)

    def fire(r, slot):
        base = r * L
        pltpu.make_async_copy(
            table_hbm.at[idx_v.at[pl.ds(base, C0)]],
            rows_v.at[slot].at[pl.ds(0, C0)], sem.at[slot]).start()
        pltpu.make_async_copy(
            table_hbm.at[idx_v.at[pl.ds(base + C0, C1)]],
            rows_v.at[slot].at[pl.ds(C0, C1)], sem.at[slot]).start()

    def drain(slot):
        pltpu.make_async_copy(
            table_hbm.at[pl.ds(0, L)], rows_v.at[slot], sem.at[slot]).wait()

    for p in range(NBUF - 1):
        fire(p, p)

    @pl.loop(0, BPW, step=NBUF)
    def _blk(r0):
        vec = jnp.zeros((16,), jnp.float32)
        for k in range(NBUF):
            r = r0 + k
            nxt = r + NBUF - 1

            @pl.when(nxt < BPW)
            def _():
                fire(nxt, (k + NBUF - 1) % NBUF)

            drain(k)

            def acc_body(l, carry):
                a0, a1 = carry
                a0 = a0 + rows_v[k, l, pl.ds(0, 16)]
                a1 = a1 + rows_v[k, l, pl.ds(16, 16)]
                return a0, a1

            zero = jnp.zeros((16,), jnp.float32)
            a0, a1 = lax.fori_loop(0, L, acc_body, (zero, zero), unroll=8)
            t = (a0 * w0 + a1 * w1) * (1.0 / L) + biasv
            vec = jnp.where(lane == k, jnp.sum(t), vec)

        out_v[pl.ds(r0, 16)] = vec

    pltpu.sync_copy(out_v, out_hbm.at[pl.ds(wid * BPW, BPW)])


def kernel(x, table, W, b):
    xf = x.reshape(-1).astype(jnp.int32)
    wb = jnp.concatenate(
        [W.reshape(-1), jnp.broadcast_to(b / 16.0, (16,))]
    ).astype(jnp.float32)

    mesh = plsc.VectorSubcoreMesh(core_axis_name="c", subcore_axis_name="s")
    cp = pltpu.CompilerParams()
    fields = pltpu.CompilerParams.__dataclass_fields__
    if "needs_layout_passes" in fields:
        cp = dataclasses.replace(cp, needs_layout_passes=False)
    if "use_tc_tiling_on_sc" in fields:
        cp = dataclasses.replace(cp, use_tc_tiling_on_sc=False)
    run = functools.partial(
        pl.kernel,
        compiler_params=cp,
        out_type=jax.ShapeDtypeStruct((B,), jnp.float32),
        mesh=mesh,
        scratch_types=[
            pltpu.VMEM((IPW,), jnp.int32),
            pltpu.VMEM((NBUF, L, D), jnp.float32),
            pltpu.VMEM((BPW,), jnp.float32),
            pltpu.VMEM((48,), jnp.float32),
            pltpu.SemaphoreType.DMA((NBUF,)),
        ],
    )(_sc_kernel)

    out = run(xf, table, wb)
    return out.reshape(B, 1)


# speedup vs baseline: 2.7046x; 1.0935x over previous
import dataclasses
import functools

import jax
import jax.numpy as jnp
from jax import lax
from jax.experimental import pallas as pl
from jax.experimental.pallas import tpu as pltpu
from jax.experimental.pallas import tpu_sc as plsc

B = 4096
L = 200
D = 32
V = 1000000
NW = 32
BPW = B // NW
IPW = BPW * L
GCH = 128
NCH = IPW // GCH

TC_BLK = 16384
TC_GRID = -(-V // TC_BLK)
VP = TC_GRID * TC_BLK


def _tw_kernel(w_ref, t_ref, o_ref):
    r = jax.lax.dot_general(
        w_ref[...], t_ref[...],
        dimension_numbers=(((1,), (1,)), ((), ())),
        preferred_element_type=jnp.float32)
    o_ref[...] = r.reshape(TC_BLK)


def _table_times_w(table, w_row):
    return pl.pallas_call(
        _tw_kernel,
        grid=(TC_GRID,),
        in_specs=[
            pl.BlockSpec((1, D), lambda i: (0, 0)),
            pl.BlockSpec((TC_BLK, D), lambda i: (i, 0)),
        ],
        out_specs=pl.BlockSpec((TC_BLK,), lambda i: (i,)),
        out_shape=jax.ShapeDtypeStruct((VP,), jnp.float32),
    )(w_row, table)


def _sc_kernel(x_hbm, tw_hbm, bias_hbm, out_hbm, idx_v, val_v, out_v,
               bias_v, sem):
    wid = lax.axis_index("s") * 2 + lax.axis_index("c")

    pltpu.sync_copy(bias_hbm, bias_v)
    pltpu.sync_copy(x_hbm.at[pl.ds(wid * IPW, IPW)], idx_v)

    @pl.loop(0, NCH)
    def _fire(c):
        pltpu.make_async_copy(
            tw_hbm.at[idx_v.at[pl.ds(c * GCH, GCH)]],
            val_v.at[pl.ds(c * GCH, GCH)], sem).start()

    pltpu.make_async_copy(tw_hbm.at[pl.ds(0, IPW)], val_v, sem).wait()

    biasv = bias_v[pl.ds(0, 16)]
    lane = lax.broadcasted_iota(jnp.int32, (16,), 0)
    scale = 1.0 / L

    @pl.loop(0, BPW // 16)
    def _grp(g):
        vec = jnp.zeros((16,), jnp.float32)
        for k in range(16):
            base = (g * 16 + k) * L

            def acc_body(j, a):
                return a + val_v[pl.ds(base + j * 16, 16)]

            a = lax.fori_loop(0, 12, acc_body, jnp.zeros((16,), jnp.float32),
                              unroll=True)
            tail = val_v[pl.ds(base + 184, 16)]
            a = a + jnp.where(lane < 8, 0.0, tail)
            t = a * scale + biasv
            vec = jnp.where(lane == k, jnp.sum(t), vec)
        out_v[pl.ds(g * 16, 16)] = vec

    pltpu.sync_copy(out_v, out_hbm.at[pl.ds(wid * BPW, BPW)])


def kernel(x, table, W, b):
    xf = x.reshape(-1).astype(jnp.int32)
    w_row = W.reshape(1, D).astype(jnp.float32)
    tw = _table_times_w(table, w_row)
    bias16 = jnp.broadcast_to(b.astype(jnp.float32) / 16.0, (16,))

    mesh = plsc.VectorSubcoreMesh(core_axis_name="c", subcore_axis_name="s")
    cp = pltpu.CompilerParams()
    fields = pltpu.CompilerParams.__dataclass_fields__
    if "needs_layout_passes" in fields:
        cp = dataclasses.replace(cp, needs_layout_passes=False)
    if "use_tc_tiling_on_sc" in fields:
        cp = dataclasses.replace(cp, use_tc_tiling_on_sc=False)
    run = functools.partial(
        pl.kernel,
        compiler_params=cp,
        out_type=jax.ShapeDtypeStruct((B,), jnp.float32),
        mesh=mesh,
        scratch_types=[
            pltpu.VMEM((IPW,), jnp.int32),
            pltpu.VMEM((IPW,), jnp.float32),
            pltpu.VMEM((BPW,), jnp.float32),
            pltpu.VMEM((16,), jnp.float32),
            pltpu.SemaphoreType.DMA,
        ],
    )(_sc_kernel)

    out = run(xf, tw, bias16)
    return out.reshape(B, 1)
